# Initial kernel scaffold; baseline (speedup 1.0000x reference)
import jax
import jax.numpy as jnp
from jax.experimental import pallas as pl


def kernel(etypes, dst, num_nodes, rel_head_emb, rel_tail_emb):
    raise NotImplementedError("write your pallas kernel here")


# speedup vs baseline: 7.9939x; 7.9939x over previous
import functools

import jax
import jax.numpy as jnp
from jax import lax
from jax.experimental import pallas as pl
from jax.experimental.pallas import tpu as pltpu
from jax.experimental.pallas import tpu_sc as plsc

NE = 320000
NN = 10000
D = 128
HD = D // 2
CW = 16
NC = 2
NS = 16
EPT = NE // NS
B = 80
NB = EPT // B
RPT = 624
TAIL = NN - NS * RPT
ZR = 48

_mesh = plsc.VectorSubcoreMesh(
    core_axis_name="c", subcore_axis_name="s", num_cores=NC, num_subcores=NS
)


@functools.partial(
    pl.kernel,
    out_type=[
        jax.ShapeDtypeStruct((NC, NN, HD), jnp.float32),
        jax.ShapeDtypeStruct((NN, CW), jnp.float32),
    ],
    mesh=_mesh,
    compiler_params=pltpu.CompilerParams(use_tc_tiling_on_sc=False),
    scratch_types=[
        pltpu.VMEM((NB, B), jnp.int32),
        pltpu.VMEM((NB, B), jnp.int32),
        pltpu.VMEM((B, HD), jnp.float32),
        pltpu.VMEM((ZR, HD), jnp.float32),
        pltpu.VMEM((ZR, CW), jnp.float32),
        pltpu.VMEM((B, CW), jnp.float32),
        pltpu.VMEM_SHARED((NN, HD), jnp.float32),
        pltpu.VMEM_SHARED((NN, CW), jnp.float32),
        pltpu.SemaphoreType.DMA,
    ],
)
def _sc_gather_scatter(
    et_hbm, dst_hbm, tlo_hbm, thi_hbm, acc_out, cnt_out,
    et_v, dst_v, rows_v, zbuf, zcbuf, ones_v, acc_sh, cnt_sh, sem,
):
    c = lax.axis_index("c")
    s = lax.axis_index("s")

    pltpu.sync_copy(et_hbm.at[s], et_v)
    pltpu.sync_copy(dst_hbm.at[s], dst_v)

    z16 = jnp.zeros((16,), jnp.float32)
    ones16 = jnp.full((16,), 1.0, jnp.float32)

    def _zrow(r, carry):
        for k in range(HD // 16):
            zbuf[r, pl.ds(k * 16, 16)] = z16
        zcbuf[r, pl.ds(0, CW)] = z16
        return carry

    lax.fori_loop(0, ZR, _zrow, None)

    def _orow(r, carry):
        ones_v[r, pl.ds(0, CW)] = ones16
        return carry

    lax.fori_loop(0, B, _orow, None)

    for t in range(RPT // ZR):
        pltpu.sync_copy(zbuf, acc_sh.at[pl.ds(s * RPT + t * ZR, ZR)])
        pltpu.sync_copy(zcbuf, cnt_sh.at[pl.ds(s * RPT + t * ZR, ZR)])

    @pl.when(s == NS - 1)
    def _zero_tail():
        pltpu.sync_copy(zbuf.at[pl.ds(0, TAIL)], acc_sh.at[pl.ds(NS * RPT, TAIL)])
        pltpu.sync_copy(zcbuf.at[pl.ds(0, TAIL)], cnt_sh.at[pl.ds(NS * RPT, TAIL)])

    plsc.subcore_barrier()

    def _block(j, carry):
        @pl.when(c == 0)
        def _g0():
            pltpu.async_copy(tlo_hbm.at[et_v.at[j]], rows_v, sem).wait()

        @pl.when(c == 1)
        def _g1():
            pltpu.async_copy(thi_hbm.at[et_v.at[j]], rows_v, sem).wait()

        pltpu.sync_copy(rows_v, acc_sh.at[dst_v.at[j]], add=True)

        @pl.when(c == 0)
        def _cnt():
            pltpu.sync_copy(ones_v, cnt_sh.at[dst_v.at[j]], add=True)

        return carry

    lax.fori_loop(0, NB, _block, None)

    plsc.subcore_barrier()

    pltpu.sync_copy(
        acc_sh.at[pl.ds(s * RPT, RPT)], acc_out.at[c, pl.ds(s * RPT, RPT)]
    )

    @pl.when(c == 0)
    def _pub_cnt():
        pltpu.sync_copy(cnt_sh.at[pl.ds(s * RPT, RPT)], cnt_out.at[pl.ds(s * RPT, RPT)])

    @pl.when(s == NS - 1)
    def _pub_tail():
        pltpu.sync_copy(
            acc_sh.at[pl.ds(NS * RPT, TAIL)], acc_out.at[c, pl.ds(NS * RPT, TAIL)]
        )

        @pl.when(c == 0)
        def _pub_cnt_tail():
            pltpu.sync_copy(
                cnt_sh.at[pl.ds(NS * RPT, TAIL)], cnt_out.at[pl.ds(NS * RPT, TAIL)]
            )


def _finish_body(acc_ref, cnt_ref, out_ref):
    sums = jnp.concatenate([acc_ref[0], acc_ref[1]], axis=1)
    cnt = cnt_ref[:, 0:1]
    out_ref[...] = jnp.where(cnt > 0, sums / jnp.maximum(cnt, 1.0), 0.0)


_FIN_R = 1000


def _finish(acc, cnt):
    return pl.pallas_call(
        _finish_body,
        grid=(NN // _FIN_R,),
        in_specs=[
            pl.BlockSpec((NC, _FIN_R, HD), lambda i: (0, i, 0)),
            pl.BlockSpec((_FIN_R, CW), lambda i: (i, 0)),
        ],
        out_specs=pl.BlockSpec((_FIN_R, D), lambda i: (i, 0)),
        out_shape=jax.ShapeDtypeStruct((NN, D), jnp.float32),
    )(acc, cnt)


def kernel(etypes, dst, num_nodes, rel_head_emb, rel_tail_emb):
    num_rel = rel_head_emb.shape[0]
    table = jnp.concatenate([rel_head_emb, rel_tail_emb], axis=0)
    tlo = table[:, :HD]
    thi = table[:, HD:]
    et = jnp.clip(etypes, 0, 2 * num_rel - 1).reshape(NS, NB, B)
    ds_ = jnp.clip(dst, 0, NN - 1).reshape(NS, NB, B)
    acc, cnt = _sc_gather_scatter(et, ds_, tlo, thi)
    return _finish(acc, cnt)


# speedup vs baseline: 16.2778x; 2.0363x over previous
import functools

import jax
import jax.numpy as jnp
from jax import lax
from jax.experimental import pallas as pl
from jax.experimental.pallas import tpu as pltpu
from jax.experimental.pallas import tpu_sc as plsc

NE = 320000
NN = 10000
D = 128
HD = D // 2
CW = 16
NC = 2
NS = 16
EPT = NE // NS
B = 80
NB = EPT // B
RPT = 624
TAIL = NN - NS * RPT
ZR = 48
NBUF = 5
LOOK = 3

_mesh = plsc.VectorSubcoreMesh(
    core_axis_name="c", subcore_axis_name="s", num_cores=NC, num_subcores=NS
)


@functools.partial(
    pl.kernel,
    out_type=[
        jax.ShapeDtypeStruct((NC, NN, HD), jnp.float32),
        jax.ShapeDtypeStruct((NC, NN, CW), jnp.float32),
    ],
    mesh=_mesh,
    compiler_params=pltpu.CompilerParams(use_tc_tiling_on_sc=False),
    scratch_types=[
        pltpu.VMEM((NB, B), jnp.int32),
        pltpu.VMEM((NB, B), jnp.int32),
        pltpu.VMEM((NBUF, B, HD), jnp.float32),
        pltpu.VMEM((ZR, HD), jnp.float32),
        pltpu.VMEM((ZR, CW), jnp.float32),
        pltpu.VMEM((B, CW), jnp.float32),
        pltpu.VMEM_SHARED((NN, HD), jnp.float32),
        pltpu.VMEM_SHARED((NN, CW), jnp.float32),
        pltpu.SemaphoreType.DMA((NBUF,)),
        pltpu.SemaphoreType.DMA((NBUF,)),
        pltpu.SemaphoreType.DMA((NBUF,)),
    ],
)
def _sc_gather_scatter(
    et_hbm, dst_hbm, tlo_hbm, thi_hbm, acc_out, cnt_out,
    et_v, dst_v, rows_v, zbuf, zcbuf, ones_v, acc_sh, cnt_sh,
    sem_g, sem_s, sem_c,
):
    c = lax.axis_index("c")
    s = lax.axis_index("s")

    pltpu.sync_copy(et_hbm.at[s], et_v)
    pltpu.sync_copy(dst_hbm.at[s], dst_v)

    z16 = jnp.zeros((16,), jnp.float32)
    half16 = jnp.full((16,), 0.5, jnp.float32)

    def _zrow(r, carry):
        for k in range(HD // 16):
            zbuf[r, pl.ds(k * 16, 16)] = z16
        zcbuf[r, pl.ds(0, CW)] = z16
        return carry

    lax.fori_loop(0, ZR, _zrow, None)

    def _orow(r, carry):
        ones_v[r, pl.ds(0, CW)] = half16
        return carry

    lax.fori_loop(0, B, _orow, None)

    for t in range(RPT // ZR):
        pltpu.sync_copy(zbuf, acc_sh.at[pl.ds(s * RPT + t * ZR, ZR)])
        pltpu.sync_copy(zcbuf, cnt_sh.at[pl.ds(s * RPT + t * ZR, ZR)])

    @pl.when(s == NS - 1)
    def _zero_tail():
        pltpu.sync_copy(zbuf.at[pl.ds(0, TAIL)], acc_sh.at[pl.ds(NS * RPT, TAIL)])
        pltpu.sync_copy(zcbuf.at[pl.ds(0, TAIL)], cnt_sh.at[pl.ds(NS * RPT, TAIL)])

    plsc.subcore_barrier()

    def _gather_start(j, b):
        @pl.when(c == 0)
        def _g0():
            pltpu.async_copy(tlo_hbm.at[et_v.at[j]], rows_v.at[b], sem_g.at[b])

        @pl.when(c == 1)
        def _g1():
            pltpu.async_copy(thi_hbm.at[et_v.at[j]], rows_v.at[b], sem_g.at[b])

    def _gather_wait(b):
        pltpu.make_async_copy(
            tlo_hbm.at[et_v.at[0]], rows_v.at[b], sem_g.at[b]
        ).wait()

    def _scatter_wait(b):
        pltpu.make_async_copy(
            rows_v.at[b], acc_sh.at[dst_v.at[0]], sem_s.at[b]
        ).wait()

    def _cnt_wait(b):
        pltpu.make_async_copy(
            ones_v, cnt_sh.at[dst_v.at[0]], sem_c.at[b]
        ).wait()

    for b in range(LOOK):
        _gather_start(b, b)

    def _group(g, carry):
        for b in range(NBUF):
            j = g * NBUF + b
            b5 = (b + LOOK) % NBUF
            _gather_wait(b)
            pltpu.async_copy(rows_v.at[b], acc_sh.at[dst_v.at[j]], sem_s.at[b], add=True)

            @pl.when(j >= NBUF)
            def _drain_cnt():
                _cnt_wait(b)

            pltpu.async_copy(ones_v, cnt_sh.at[dst_v.at[j]], sem_c.at[b], add=True)

            @pl.when(j >= NBUF - LOOK)
            def _drain_scatter():
                _scatter_wait(b5)

            @pl.when(j + LOOK < NB)
            def _next_gather():
                _gather_start(j + LOOK, b5)

        return carry

    lax.fori_loop(0, NB // NBUF, _group, None)

    for b in range(LOOK, NBUF):
        _scatter_wait(b)
    for b in range(NBUF):
        _cnt_wait(b)

    plsc.subcore_barrier()

    pltpu.sync_copy(
        acc_sh.at[pl.ds(s * RPT, RPT)], acc_out.at[c, pl.ds(s * RPT, RPT)]
    )

    pltpu.sync_copy(
        cnt_sh.at[pl.ds(s * RPT, RPT)], cnt_out.at[c, pl.ds(s * RPT, RPT)]
    )

    @pl.when(s == NS - 1)
    def _pub_tail():
        pltpu.sync_copy(
            acc_sh.at[pl.ds(NS * RPT, TAIL)], acc_out.at[c, pl.ds(NS * RPT, TAIL)]
        )
        pltpu.sync_copy(
            cnt_sh.at[pl.ds(NS * RPT, TAIL)], cnt_out.at[c, pl.ds(NS * RPT, TAIL)]
        )


def _finish_body(acc_ref, cnt_ref, out_ref):
    sums = jnp.concatenate([acc_ref[0], acc_ref[1]], axis=1)
    cnt = (cnt_ref[0] + cnt_ref[1])[:, 0:1]
    out_ref[...] = jnp.where(cnt > 0, sums / jnp.maximum(cnt, 1.0), 0.0)


_FIN_R = 1000


def _finish(acc, cnt):
    return pl.pallas_call(
        _finish_body,
        grid=(NN // _FIN_R,),
        in_specs=[
            pl.BlockSpec((NC, _FIN_R, HD), lambda i: (0, i, 0)),
            pl.BlockSpec((NC, _FIN_R, CW), lambda i: (0, i, 0)),
        ],
        out_specs=pl.BlockSpec((_FIN_R, D), lambda i: (i, 0)),
        out_shape=jax.ShapeDtypeStruct((NN, D), jnp.float32),
    )(acc, cnt)


def kernel(etypes, dst, num_nodes, rel_head_emb, rel_tail_emb):
    num_rel = rel_head_emb.shape[0]
    table = jnp.concatenate([rel_head_emb, rel_tail_emb], axis=0)
    tlo = table[:, :HD]
    thi = table[:, HD:]
    et = jnp.clip(etypes, 0, 2 * num_rel - 1).reshape(NS, NB, B)
    ds_ = jnp.clip(dst, 0, NN - 1).reshape(NS, NB, B)
    acc, cnt = _sc_gather_scatter(et, ds_, tlo, thi)
    return _finish(acc, cnt)


# speedup vs baseline: 17.0365x; 1.0466x over previous
import functools

import jax
import jax.numpy as jnp
from jax import lax
from jax.experimental import pallas as pl
from jax.experimental.pallas import tpu as pltpu
from jax.experimental.pallas import tpu_sc as plsc

NE = 320000
NN = 10000
D = 128
HD = D // 2
CW = 16
NC = 2
NS = 16
EPT = NE // NS
B = 80
NB = EPT // B
RPT = 624
TAIL = NN - NS * RPT
ZR = 48
NBUF = 5
LOOK = 3

_mesh = plsc.VectorSubcoreMesh(
    core_axis_name="c", subcore_axis_name="s", num_cores=NC, num_subcores=NS
)


@functools.partial(
    pl.kernel,
    out_type=[
        jax.ShapeDtypeStruct((NC, NN, HD), jnp.float32),
        jax.ShapeDtypeStruct((NC, NN, CW), jnp.float32),
    ],
    mesh=_mesh,
    compiler_params=pltpu.CompilerParams(use_tc_tiling_on_sc=False),
    scratch_types=[
        pltpu.VMEM((NB, B), jnp.int32),
        pltpu.VMEM((NB, B), jnp.int32),
        pltpu.VMEM((NBUF, B, HD), jnp.float32),
        pltpu.VMEM((ZR, HD), jnp.float32),
        pltpu.VMEM((ZR, CW), jnp.float32),
        pltpu.VMEM((B, CW), jnp.float32),
        pltpu.VMEM_SHARED((NN, HD), jnp.float32),
        pltpu.VMEM_SHARED((NN, CW), jnp.float32),
        pltpu.SemaphoreType.DMA((NBUF,)),
        pltpu.SemaphoreType.DMA((NBUF,)),
        pltpu.SemaphoreType.DMA((NBUF,)),
    ],
)
def _sc_gather_scatter(
    et_hbm, dst_hbm, tlo_hbm, thi_hbm, acc_out, cnt_out,
    et_v, dst_v, rows_v, zbuf, zcbuf, ones_v, acc_sh, cnt_sh,
    sem_g, sem_s, sem_c,
):
    c = lax.axis_index("c")
    s = lax.axis_index("s")

    pltpu.sync_copy(et_hbm.at[s], et_v)
    pltpu.sync_copy(dst_hbm.at[s], dst_v)

    z16 = jnp.zeros((16,), jnp.float32)
    ones16 = jnp.full((16,), 1.0, jnp.float32)

    def _zrow(r, carry):
        for k in range(HD // 16):
            zbuf[r, pl.ds(k * 16, 16)] = z16
        zcbuf[r, pl.ds(0, CW)] = z16
        return carry

    lax.fori_loop(0, ZR, _zrow, None)

    def _orow(r, carry):
        ones_v[r, pl.ds(0, CW)] = ones16
        return carry

    lax.fori_loop(0, B, _orow, None)

    for t in range(RPT // ZR):
        pltpu.sync_copy(zbuf, acc_sh.at[pl.ds(s * RPT + t * ZR, ZR)])
        pltpu.sync_copy(zcbuf, cnt_sh.at[pl.ds(s * RPT + t * ZR, ZR)])

    @pl.when(s == NS - 1)
    def _zero_tail():
        pltpu.sync_copy(zbuf.at[pl.ds(0, TAIL)], acc_sh.at[pl.ds(NS * RPT, TAIL)])
        pltpu.sync_copy(zcbuf.at[pl.ds(0, TAIL)], cnt_sh.at[pl.ds(NS * RPT, TAIL)])

    plsc.subcore_barrier()

    def _gather_start(j, b):
        @pl.when(c == 0)
        def _g0():
            pltpu.async_copy(tlo_hbm.at[et_v.at[j]], rows_v.at[b], sem_g.at[b])

        @pl.when(c == 1)
        def _g1():
            pltpu.async_copy(thi_hbm.at[et_v.at[j]], rows_v.at[b], sem_g.at[b])

    def _gather_wait(b):
        pltpu.make_async_copy(
            tlo_hbm.at[et_v.at[0]], rows_v.at[b], sem_g.at[b]
        ).wait()

    def _scatter_wait(b):
        pltpu.make_async_copy(
            rows_v.at[b], acc_sh.at[dst_v.at[0]], sem_s.at[b]
        ).wait()

    def _cnt_wait(b):
        pltpu.make_async_copy(
            ones_v, cnt_sh.at[dst_v.at[0]], sem_c.at[b]
        ).wait()

    for b in range(LOOK):
        _gather_start(b, b)

    def _group(g, carry):
        for b in range(NBUF):
            j = g * NBUF + b
            b5 = (b + LOOK) % NBUF
            _gather_wait(b)
            pltpu.async_copy(rows_v.at[b], acc_sh.at[dst_v.at[j]], sem_s.at[b], add=True)

            @pl.when(j % 2 == c)
            def _count_block():
                @pl.when(j >= 2 * NBUF)
                def _drain_cnt():
                    _cnt_wait(b)

                pltpu.async_copy(
                    ones_v, cnt_sh.at[dst_v.at[j]], sem_c.at[b], add=True
                )

            @pl.when(j >= NBUF - LOOK)
            def _drain_scatter():
                _scatter_wait(b5)

            @pl.when(j + LOOK < NB)
            def _next_gather():
                _gather_start(j + LOOK, b5)

        return carry

    lax.fori_loop(0, NB // NBUF, _group, None)

    for b in range(LOOK, NBUF):
        _scatter_wait(b)
    for b in range(NBUF):
        _cnt_wait(b)

    plsc.subcore_barrier()

    pltpu.sync_copy(
        acc_sh.at[pl.ds(s * RPT, RPT)], acc_out.at[c, pl.ds(s * RPT, RPT)]
    )

    pltpu.sync_copy(
        cnt_sh.at[pl.ds(s * RPT, RPT)], cnt_out.at[c, pl.ds(s * RPT, RPT)]
    )

    @pl.when(s == NS - 1)
    def _pub_tail():
        pltpu.sync_copy(
            acc_sh.at[pl.ds(NS * RPT, TAIL)], acc_out.at[c, pl.ds(NS * RPT, TAIL)]
        )
        pltpu.sync_copy(
            cnt_sh.at[pl.ds(NS * RPT, TAIL)], cnt_out.at[c, pl.ds(NS * RPT, TAIL)]
        )


def _finish_body(acc_ref, cnt_ref, out_ref):
    sums = jnp.concatenate([acc_ref[0], acc_ref[1]], axis=1)
    cnt = (cnt_ref[0] + cnt_ref[1])[:, 0:1]
    out_ref[...] = jnp.where(cnt > 0, sums / jnp.maximum(cnt, 1.0), 0.0)


_FIN_R = 1000


def _finish(acc, cnt):
    return pl.pallas_call(
        _finish_body,
        grid=(NN // _FIN_R,),
        in_specs=[
            pl.BlockSpec((NC, _FIN_R, HD), lambda i: (0, i, 0)),
            pl.BlockSpec((NC, _FIN_R, CW), lambda i: (0, i, 0)),
        ],
        out_specs=pl.BlockSpec((_FIN_R, D), lambda i: (i, 0)),
        out_shape=jax.ShapeDtypeStruct((NN, D), jnp.float32),
    )(acc, cnt)


def kernel(etypes, dst, num_nodes, rel_head_emb, rel_tail_emb):
    num_rel = rel_head_emb.shape[0]
    table = jnp.concatenate([rel_head_emb, rel_tail_emb], axis=0)
    tlo = table[:, :HD]
    thi = table[:, HD:]
    del num_rel
    et = etypes.reshape(NS, NB, B)
    ds_ = dst.reshape(NS, NB, B)
    acc, cnt = _sc_gather_scatter(et, ds_, tlo, thi)
    return _finish(acc, cnt)


# speedup vs baseline: 18.1810x; 1.0672x over previous
import functools

import jax
import jax.numpy as jnp
from jax import lax
from jax.experimental import pallas as pl
from jax.experimental.pallas import tpu as pltpu
from jax.experimental.pallas import tpu_sc as plsc

NE = 320000
NN = 10000
D = 128
HD = D // 2
CW = 16
NC = 2
NS = 16
EPT = NE // NS
B = 80
NB = EPT // B
RPT = 624
TAIL = NN - NS * RPT
ZR = 48
NBUF = 5
LOOK = 3

_mesh = plsc.VectorSubcoreMesh(
    core_axis_name="c", subcore_axis_name="s", num_cores=NC, num_subcores=NS
)


@functools.partial(
    pl.kernel,
    out_type=jax.ShapeDtypeStruct((NN, D), jnp.float32),
    mesh=_mesh,
    compiler_params=pltpu.CompilerParams(use_tc_tiling_on_sc=False),
    scratch_types=[
        pltpu.VMEM((NB, B), jnp.int32),
        pltpu.VMEM((NB, B), jnp.int32),
        pltpu.VMEM((NBUF, B, HD), jnp.float32),
        pltpu.VMEM((ZR, HD), jnp.float32),
        pltpu.VMEM((ZR, CW), jnp.float32),
        pltpu.VMEM((B, CW), jnp.float32),
        pltpu.VMEM_SHARED((NN, HD), jnp.float32),
        pltpu.VMEM_SHARED((NN, CW), jnp.float32),
        pltpu.SemaphoreType.DMA((NBUF,)),
        pltpu.SemaphoreType.DMA((NBUF,)),
        pltpu.SemaphoreType.DMA((NBUF,)),
    ],
)
def _sc_gather_scatter(
    et_hbm, dst_hbm, tlo_hbm, thi_hbm, out_hbm,
    et_v, dst_v, rows_v, zbuf, zcbuf, ones_v, acc_sh, cnt_sh,
    sem_g, sem_s, sem_c,
):
    c = lax.axis_index("c")
    s = lax.axis_index("s")

    pltpu.sync_copy(et_hbm.at[s], et_v)
    pltpu.sync_copy(dst_hbm.at[s], dst_v)

    z16 = jnp.zeros((16,), jnp.float32)
    ones16 = jnp.full((16,), 1.0, jnp.float32)

    def _zrow(r, carry):
        for k in range(HD // 16):
            zbuf[r, pl.ds(k * 16, 16)] = z16
        zcbuf[r, pl.ds(0, CW)] = z16
        return carry

    lax.fori_loop(0, ZR, _zrow, None)

    def _orow(r, carry):
        ones_v[r, pl.ds(0, CW)] = ones16
        return carry

    lax.fori_loop(0, B, _orow, None)

    for t in range(RPT // ZR):
        pltpu.sync_copy(zbuf, acc_sh.at[pl.ds(s * RPT + t * ZR, ZR)])
        pltpu.sync_copy(zcbuf, cnt_sh.at[pl.ds(s * RPT + t * ZR, ZR)])

    @pl.when(s == NS - 1)
    def _zero_tail():
        pltpu.sync_copy(zbuf.at[pl.ds(0, TAIL)], acc_sh.at[pl.ds(NS * RPT, TAIL)])
        pltpu.sync_copy(zcbuf.at[pl.ds(0, TAIL)], cnt_sh.at[pl.ds(NS * RPT, TAIL)])

    plsc.subcore_barrier()

    def _gather_start(j, b):
        @pl.when(c == 0)
        def _g0():
            pltpu.async_copy(tlo_hbm.at[et_v.at[j]], rows_v.at[b], sem_g.at[b])

        @pl.when(c == 1)
        def _g1():
            pltpu.async_copy(thi_hbm.at[et_v.at[j]], rows_v.at[b], sem_g.at[b])

    def _gather_wait(b):
        pltpu.make_async_copy(
            tlo_hbm.at[et_v.at[0]], rows_v.at[b], sem_g.at[b]
        ).wait()

    def _scatter_wait(b):
        pltpu.make_async_copy(
            rows_v.at[b], acc_sh.at[dst_v.at[0]], sem_s.at[b]
        ).wait()

    def _cnt_wait(b):
        pltpu.make_async_copy(
            ones_v, cnt_sh.at[dst_v.at[0]], sem_c.at[b]
        ).wait()

    for b in range(LOOK):
        _gather_start(b, b)

    def _group(g, carry):
        for b in range(NBUF):
            j = g * NBUF + b
            b5 = (b + LOOK) % NBUF
            _gather_wait(b)
            pltpu.async_copy(rows_v.at[b], acc_sh.at[dst_v.at[j]], sem_s.at[b], add=True)

            @pl.when(j >= NBUF)
            def _drain_cnt():
                _cnt_wait(b)

            pltpu.async_copy(
                ones_v, cnt_sh.at[dst_v.at[j]], sem_c.at[b], add=True
            )

            @pl.when(j >= NBUF - LOOK)
            def _drain_scatter():
                _scatter_wait(b5)

            @pl.when(j + LOOK < NB)
            def _next_gather():
                _gather_start(j + LOOK, b5)

        return carry

    lax.fori_loop(0, NB // NBUF, _group, None)

    for b in range(LOOK, NBUF):
        _scatter_wait(b)
    for b in range(NBUF):
        _cnt_wait(b)

    plsc.subcore_barrier()

    def _finish_rows(row0, nrows):
        pltpu.sync_copy(acc_sh.at[pl.ds(row0, nrows)], zbuf.at[pl.ds(0, nrows)])
        pltpu.sync_copy(cnt_sh.at[pl.ds(row0, nrows)], zcbuf.at[pl.ds(0, nrows)])

        def _frow(r, carry):
            cnt16 = zcbuf[r, pl.ds(0, CW)]
            recip = jnp.where(cnt16 > 0.0, 1.0 / jnp.maximum(cnt16, 1.0), 0.0)
            for k in range(HD // 16):
                v = zbuf[r, pl.ds(k * 16, 16)]
                zbuf[r, pl.ds(k * 16, 16)] = v * recip
            return carry

        lax.fori_loop(0, nrows, _frow, None)
        pltpu.sync_copy(
            zbuf.at[pl.ds(0, nrows)],
            out_hbm.at[pl.ds(row0, nrows), pl.ds(c * HD, HD)],
        )

    def _fin_chunk(t, carry):
        _finish_rows(s * RPT + t * ZR, ZR)
        return carry

    lax.fori_loop(0, RPT // ZR, _fin_chunk, None)

    @pl.when(s == NS - 1)
    def _fin_tail():
        _finish_rows(NS * RPT, TAIL)


def kernel(etypes, dst, num_nodes, rel_head_emb, rel_tail_emb):
    num_rel = rel_head_emb.shape[0]
    table = jnp.concatenate([rel_head_emb, rel_tail_emb], axis=0)
    tlo = table[:, :HD]
    thi = table[:, HD:]
    del num_rel
    et = etypes.reshape(NS, NB, B)
    ds_ = dst.reshape(NS, NB, B)
    return _sc_gather_scatter(et, ds_, tlo, thi)
